# baseline (speedup 1.0000x reference)
import functools

import jax
import jax.numpy as jnp
from jax import lax
from jax.experimental import pallas as pl
from jax.experimental.pallas import tpu as pltpu
from jax.experimental.pallas import tpu_sc as plsc

_TAU = 1e-3
_R = 128


def _stage1(x, wt):
    B, D = x.shape
    K = wt.shape[1]
    BLK = 256

    def body(x_ref, wt_ref, idx_ref, gap_ref):
        xb = x_ref[...]
        wtb = wt_ref[...]
        s = lax.dot_general(
            xb, wtb, (((1,), (0,)), ((), ())),
            preferred_element_type=jnp.float32,
            precision=lax.Precision.HIGHEST,
        )
        wsq = jnp.sum(wtb * wtb, axis=0)
        d = wsq[None, :] - 2.0 * s
        m1 = jnp.min(d, axis=1)
        am = jnp.argmin(d, axis=1).astype(jnp.int32)
        kio = lax.broadcasted_iota(jnp.int32, d.shape, 1)
        masked = jnp.where(kio == am[:, None], jnp.inf, d)
        m2 = jnp.min(masked, axis=1)
        idx_ref[...] = am
        gap_ref[...] = m2 - m1

    return pl.pallas_call(
        body,
        grid=(B // BLK,),
        in_specs=[
            pl.BlockSpec((BLK, D), lambda i: (i, 0)),
            pl.BlockSpec((D, K), lambda i: (0, 0)),
        ],
        out_specs=[
            pl.BlockSpec((BLK,), lambda i: (i,)),
            pl.BlockSpec((BLK,), lambda i: (i,)),
        ],
        out_shape=[
            jax.ShapeDtypeStruct((B,), jnp.int32),
            jax.ShapeDtypeStruct((B,), jnp.float32),
        ],
    )(x, wt)


def _stage2(xta, wt):
    D, R = xta.shape
    K = wt.shape[1]
    KB = 128
    C = D // 8

    def body(xta_ref, wt_ref, out_ref, bv_ref, bi_ref):
        kk = pl.program_id(0)

        @pl.when(kk == 0)
        def _init():
            bv_ref[...] = jnp.full((1, R), jnp.inf, jnp.float32)
            bi_ref[...] = jnp.zeros((1, R), jnp.int32)

        for kg in range(KB // 8):

            U = 25

            def cbody(ci, accs):
                out = list(accs)
                for u in range(U):
                    c8 = 8 * U * ci + 8 * u
                    xt = xta_ref[pl.ds(c8, 8), :]
                    w8 = wt_ref[pl.ds(c8, 8), kg * 8:kg * 8 + 8]
                    for j in range(8):
                        df = xt - w8[:, j:j + 1]
                        sq = df * df
                        t1 = sq + pltpu.roll(sq, 4, 0)
                        t2 = t1 + pltpu.roll(t1, 2, 0)
                        t3 = t2 + pltpu.roll(t2, 1, 0)
                        out[j] = out[j] + t3
                return tuple(out)

            accs = lax.fori_loop(
                0, C // U, cbody,
                tuple(jnp.zeros((8, R), jnp.float32) for _ in range(8)))

            for j in range(8):
                kglob = kk * KB + kg * 8 + j
                dk = accs[j][0:1, :]
                better = dk < bv_ref[...]
                bv_ref[...] = jnp.where(better, dk, bv_ref[...])
                bi_ref[...] = jnp.where(better, kglob, bi_ref[...])

        @pl.when(kk == pl.num_programs(0) - 1)
        def _fin():
            out_ref[...] = jnp.broadcast_to(bi_ref[...], (8, R))

    return pl.pallas_call(
        body,
        grid=(K // KB,),
        in_specs=[
            pl.BlockSpec((D, R), lambda kk: (0, 0)),
            pl.BlockSpec((D, KB), lambda kk: (0, kk)),
        ],
        out_specs=pl.BlockSpec((8, R), lambda kk: (0, 0)),
        out_shape=jax.ShapeDtypeStruct((8, R), jnp.int32),
        scratch_shapes=[
            pltpu.VMEM((1, R), jnp.float32),
            pltpu.VMEM((1, R), jnp.int32),
        ],
    )(xta, wt)


def _sc_gather_rows(table, idx):
    K, Dp = table.shape
    B = idx.shape[0]
    info = plsc.get_sparse_core_info()
    NC, NS = info.num_cores, info.num_subcores
    NW = NC * NS
    b_per_w = max(8, B // NW)
    nw_used = B // b_per_w
    mesh = plsc.VectorSubcoreMesh(core_axis_name="c", subcore_axis_name="s")

    @functools.partial(
        pl.kernel,
        mesh=mesh,
        out_type=jax.ShapeDtypeStruct((B, Dp), jnp.float32),
        scratch_types=[
            pltpu.VMEM((b_per_w,), jnp.int32),
            pltpu.VMEM((b_per_w, Dp), jnp.float32),
            pltpu.SemaphoreType.DMA,
        ],
    )
    def k(table_hbm, idx_hbm, out_hbm, idx_v, rows_v, sem):
        wid = lax.axis_index("s") * NC + lax.axis_index("c")

        @pl.when(wid < nw_used)
        def _():
            base = wid * b_per_w
            pltpu.sync_copy(idx_hbm.at[pl.ds(base, b_per_w)], idx_v)
            pltpu.async_copy(table_hbm.at[idx_v], rows_v, sem).wait()
            pltpu.sync_copy(rows_v, out_hbm.at[pl.ds(base, b_per_w)])

    return k(table, idx)


def _tc_slice(a, D):
    B, Dp = a.shape
    BLK = 256

    def body(a_ref, o_ref):
        o_ref[...] = a_ref[:, :D]

    return pl.pallas_call(
        body,
        grid=(B // BLK,),
        in_specs=[pl.BlockSpec((BLK, Dp), lambda i: (i, 0))],
        out_specs=pl.BlockSpec((BLK, D), lambda i: (i, 0)),
        out_shape=jax.ShapeDtypeStruct((B, D), jnp.float32),
    )(a)


def kernel(encoder_embedding, W):
    x, w = encoder_embedding, W
    B, D = x.shape
    K = w.shape[0]
    Dp = -(-D // 128) * 128

    wt = w.T
    xp = jnp.pad(x, ((0, 0), (0, Dp - D)))
    wp = jnp.pad(w, ((0, 0), (0, Dp - D)))

    idx, gap = _stage1(x, wt)
    amb = jnp.nonzero(gap < _TAU, size=_R, fill_value=0)[0].astype(jnp.int32)
    xa = _sc_gather_rows(xp, amb)
    xta = xa[:, :D].T
    fix = _stage2(xta, wt)[0]
    idx_full = idx.at[amb].set(fix)
    out = _sc_gather_rows(wp, idx_full)
    return _tc_slice(out, D)


# speedup vs baseline: 1.0611x; 1.0611x over previous
import functools

import jax
import jax.numpy as jnp
from jax import lax
from jax.experimental import pallas as pl
from jax.experimental.pallas import tpu as pltpu
from jax.experimental.pallas import tpu_sc as plsc

_TAU = 1e-3
_R = 128


def _stage1(x, wt):
    B, D = x.shape
    K = wt.shape[1]
    BLK = 256

    def body(x_ref, wt_ref, idx_ref, gap_ref):
        xb = x_ref[...]
        wtb = wt_ref[...]
        s = lax.dot_general(
            xb, wtb, (((1,), (0,)), ((), ())),
            preferred_element_type=jnp.float32,
            precision=lax.Precision.HIGHEST,
        )
        wsq = jnp.sum(wtb * wtb, axis=0)
        d = wsq[None, :] - 2.0 * s
        m1 = jnp.min(d, axis=1)
        am = jnp.argmin(d, axis=1).astype(jnp.int32)
        kio = lax.broadcasted_iota(jnp.int32, d.shape, 1)
        masked = jnp.where(kio == am[:, None], jnp.inf, d)
        m2 = jnp.min(masked, axis=1)
        idx_ref[...] = am
        gap_ref[...] = m2 - m1

    return pl.pallas_call(
        body,
        grid=(B // BLK,),
        in_specs=[
            pl.BlockSpec((BLK, D), lambda i: (i, 0)),
            pl.BlockSpec((D, K), lambda i: (0, 0)),
        ],
        out_specs=[
            pl.BlockSpec((BLK,), lambda i: (i,)),
            pl.BlockSpec((BLK,), lambda i: (i,)),
        ],
        out_shape=[
            jax.ShapeDtypeStruct((B,), jnp.int32),
            jax.ShapeDtypeStruct((B,), jnp.float32),
        ],
    )(x, wt)


def _stage2(xta, wt, amb, idx):
    D, R = xta.shape
    K = wt.shape[1]
    B = idx.shape[0]
    KB = 128
    C = D // 8

    def body(xta_ref, wt_ref, amb_ref, idx_ref, out_ref, bv_ref, bi_ref):
        kk = pl.program_id(0)

        @pl.when(kk == 0)
        def _init():
            bv_ref[...] = jnp.full((1, R), jnp.inf, jnp.float32)
            bi_ref[...] = jnp.zeros((1, R), jnp.int32)

        for kg in range(KB // 8):

            U = 25

            def cbody(ci, accs):
                out = list(accs)
                for u in range(U):
                    c8 = 8 * U * ci + 8 * u
                    xt = xta_ref[pl.ds(c8, 8), :]
                    w8 = wt_ref[pl.ds(c8, 8), kg * 8:kg * 8 + 8]
                    for j in range(8):
                        df = xt - w8[:, j:j + 1]
                        sq = df * df
                        t1 = sq + pltpu.roll(sq, 4, 0)
                        t2 = t1 + pltpu.roll(t1, 2, 0)
                        t3 = t2 + pltpu.roll(t2, 1, 0)
                        out[j] = out[j] + t3
                return tuple(out)

            accs = lax.fori_loop(
                0, C // U, cbody,
                tuple(jnp.zeros((8, R), jnp.float32) for _ in range(8)))

            for j in range(8):
                kglob = kk * KB + kg * 8 + j
                dk = accs[j][0:1, :]
                better = dk < bv_ref[...]
                bv_ref[...] = jnp.where(better, dk, bv_ref[...])
                bi_ref[...] = jnp.where(better, kglob, bi_ref[...])

        @pl.when(kk == pl.num_programs(0) - 1)
        def _fin():
            ambv = amb_ref[...]
            onehot = (lax.broadcasted_iota(jnp.int32, (R, B), 1)
                      == ambv[:, None]).astype(jnp.float32)
            fixf = bi_ref[...].astype(jnp.float32)
            num = lax.dot_general(fixf, onehot, (((1,), (0,)), ((), ())),
                                  preferred_element_type=jnp.float32)
            den = lax.dot_general(jnp.ones((1, R), jnp.float32), onehot,
                                  (((1,), (0,)), ((), ())),
                                  preferred_element_type=jnp.float32)
            fixd = num / jnp.maximum(den, 1.0)
            out_ref[...] = jnp.where(den[0] >= 1.0,
                                     fixd[0].astype(jnp.int32), idx_ref[...])

    return pl.pallas_call(
        body,
        grid=(K // KB,),
        in_specs=[
            pl.BlockSpec((D, R), lambda kk: (0, 0)),
            pl.BlockSpec((D, KB), lambda kk: (0, kk)),
            pl.BlockSpec((R,), lambda kk: (0,)),
            pl.BlockSpec((B,), lambda kk: (0,)),
        ],
        out_specs=pl.BlockSpec((B,), lambda kk: (0,)),
        out_shape=jax.ShapeDtypeStruct((B,), jnp.int32),
        scratch_shapes=[
            pltpu.VMEM((1, R), jnp.float32),
            pltpu.VMEM((1, R), jnp.int32),
        ],
    )(xta, wt, amb, idx)


def _sc_gather_rows(table, idx):
    K, Dp = table.shape
    B = idx.shape[0]
    info = plsc.get_sparse_core_info()
    NC, NS = info.num_cores, info.num_subcores
    NW = NC * NS
    b_per_w = max(8, B // NW)
    nw_used = B // b_per_w
    mesh = plsc.VectorSubcoreMesh(core_axis_name="c", subcore_axis_name="s")

    @functools.partial(
        pl.kernel,
        mesh=mesh,
        out_type=jax.ShapeDtypeStruct((B, Dp), jnp.float32),
        scratch_types=[
            pltpu.VMEM((b_per_w,), jnp.int32),
            pltpu.VMEM((b_per_w, Dp), jnp.float32),
            pltpu.SemaphoreType.DMA,
        ],
    )
    def k(table_hbm, idx_hbm, out_hbm, idx_v, rows_v, sem):
        wid = lax.axis_index("s") * NC + lax.axis_index("c")

        @pl.when(wid < nw_used)
        def _():
            base = wid * b_per_w
            pltpu.sync_copy(idx_hbm.at[pl.ds(base, b_per_w)], idx_v)
            pltpu.async_copy(table_hbm.at[idx_v], rows_v, sem).wait()
            pltpu.sync_copy(rows_v, out_hbm.at[pl.ds(base, b_per_w)])

    return k(table, idx)


def kernel(encoder_embedding, W):
    x, w = encoder_embedding, W
    B, D = x.shape
    K = w.shape[0]
    Dp = -(-D // 128) * 128

    wt = w.T
    xp = jnp.pad(x, ((0, 0), (0, Dp - D)))
    wp = jnp.pad(w, ((0, 0), (0, Dp - D)))

    idx, gap = _stage1(x, wt)
    amb = jnp.nonzero(gap < _TAU, size=_R, fill_value=0)[0].astype(jnp.int32)
    xa = _sc_gather_rows(xp, amb)
    xta = xa[:, :D].T
    idx_full = _stage2(xta, wt, amb, idx)
    out = _sc_gather_rows(wp, idx_full)
    return out[:, :D]


# speedup vs baseline: 1.0619x; 1.0007x over previous
import functools

import jax
import jax.numpy as jnp
from jax import lax
from jax.experimental import pallas as pl
from jax.experimental.pallas import tpu as pltpu
from jax.experimental.pallas import tpu_sc as plsc

_TAU = 1e-3
_R = 128


def _stage1(x, wt):
    B, D = x.shape
    K = wt.shape[1]
    BLK = 256

    def body(x_ref, wt_ref, idx_ref, gap_ref):
        xb = x_ref[...]
        wtb = wt_ref[...]
        s = lax.dot_general(
            xb, wtb, (((1,), (0,)), ((), ())),
            preferred_element_type=jnp.float32,
            precision=lax.Precision.HIGHEST,
        )
        wsq = jnp.sum(wtb * wtb, axis=0)
        d = wsq[None, :] - 2.0 * s
        m1 = jnp.min(d, axis=1)
        am = jnp.argmin(d, axis=1).astype(jnp.int32)
        kio = lax.broadcasted_iota(jnp.int32, d.shape, 1)
        masked = jnp.where(kio == am[:, None], jnp.inf, d)
        m2 = jnp.min(masked, axis=1)
        idx_ref[...] = am
        gap_ref[...] = m2 - m1

    return pl.pallas_call(
        body,
        grid=(B // BLK,),
        in_specs=[
            pl.BlockSpec((BLK, D), lambda i: (i, 0)),
            pl.BlockSpec((D, K), lambda i: (0, 0)),
        ],
        out_specs=[
            pl.BlockSpec((BLK,), lambda i: (i,)),
            pl.BlockSpec((BLK,), lambda i: (i,)),
        ],
        out_shape=[
            jax.ShapeDtypeStruct((B,), jnp.int32),
            jax.ShapeDtypeStruct((B,), jnp.float32),
        ],
    )(x, wt)


def _stage2(xta, wt, amb, idx):
    D, R = xta.shape
    K = wt.shape[1]
    B = idx.shape[0]
    KB = 128
    C = D // 8

    def body(xta_ref, wt_ref, amb_ref, idx_ref, out_ref, bv_ref, bi_ref):
        kk = pl.program_id(0)

        @pl.when(kk == 0)
        def _init():
            bv_ref[...] = jnp.full((1, R), jnp.inf, jnp.float32)
            bi_ref[...] = jnp.zeros((1, R), jnp.int32)

        for kg in range(KB // 8):

            U = 25

            def cbody(ci, accs):
                out = list(accs)
                for u in range(U):
                    c8 = 8 * U * ci + 8 * u
                    xt = xta_ref[pl.ds(c8, 8), :]
                    w8 = wt_ref[pl.ds(c8, 8), kg * 8:kg * 8 + 8]
                    for j in range(8):
                        df = xt - w8[:, j:j + 1]
                        sq = df * df
                        t1 = sq + pltpu.roll(sq, 4, 0)
                        t2 = t1 + pltpu.roll(t1, 2, 0)
                        t3 = t2 + pltpu.roll(t2, 1, 0)
                        out[j] = out[j] + t3
                return tuple(out)

            accs = lax.fori_loop(
                0, C // U, cbody,
                tuple(jnp.zeros((8, R), jnp.float32) for _ in range(8)))

            for j in range(8):
                kglob = kk * KB + kg * 8 + j
                dk = accs[j][0:1, :]
                better = dk < bv_ref[...]
                bv_ref[...] = jnp.where(better, dk, bv_ref[...])
                bi_ref[...] = jnp.where(better, kglob, bi_ref[...])

        @pl.when(kk == pl.num_programs(0) - 1)
        def _fin():
            ambv = amb_ref[...]
            onehot = (lax.broadcasted_iota(jnp.int32, (R, B), 1)
                      == ambv[:, None]).astype(jnp.float32)
            fixf = bi_ref[...].astype(jnp.float32)
            num = lax.dot_general(fixf, onehot, (((1,), (0,)), ((), ())),
                                  preferred_element_type=jnp.float32)
            den = lax.dot_general(jnp.ones((1, R), jnp.float32), onehot,
                                  (((1,), (0,)), ((), ())),
                                  preferred_element_type=jnp.float32)
            fixd = num / jnp.maximum(den, 1.0)
            out_ref[...] = jnp.where(den[0] >= 1.0,
                                     fixd[0].astype(jnp.int32), idx_ref[...])

    return pl.pallas_call(
        body,
        grid=(K // KB,),
        in_specs=[
            pl.BlockSpec((D, R), lambda kk: (0, 0)),
            pl.BlockSpec((D, KB), lambda kk: (0, kk)),
            pl.BlockSpec((R,), lambda kk: (0,)),
            pl.BlockSpec((B,), lambda kk: (0,)),
        ],
        out_specs=pl.BlockSpec((B,), lambda kk: (0,)),
        out_shape=jax.ShapeDtypeStruct((B,), jnp.int32),
        scratch_shapes=[
            pltpu.VMEM((1, R), jnp.float32),
            pltpu.VMEM((1, R), jnp.int32),
        ],
    )(xta, wt, amb, idx)


def _sc_gather_rows(table, idx):
    K, Dp = table.shape
    B = idx.shape[0]
    info = plsc.get_sparse_core_info()
    NC, NS = info.num_cores, info.num_subcores
    NW = NC * NS
    b_per_w = max(8, B // NW)
    nw_used = B // b_per_w
    mesh = plsc.VectorSubcoreMesh(core_axis_name="c", subcore_axis_name="s")

    @functools.partial(
        pl.kernel,
        mesh=mesh,
        out_type=jax.ShapeDtypeStruct((B, Dp), jnp.float32),
        scratch_types=[
            pltpu.VMEM((b_per_w,), jnp.int32),
            pltpu.VMEM((b_per_w, Dp), jnp.float32),
            pltpu.SemaphoreType.DMA,
        ],
    )
    def k(table_hbm, idx_hbm, out_hbm, idx_v, rows_v, sem):
        wid = lax.axis_index("s") * NC + lax.axis_index("c")

        @pl.when(wid < nw_used)
        def _():
            base = wid * b_per_w
            pltpu.sync_copy(idx_hbm.at[pl.ds(base, b_per_w)], idx_v)
            pltpu.async_copy(table_hbm.at[idx_v], rows_v, sem).wait()
            pltpu.sync_copy(rows_v, out_hbm.at[pl.ds(base, b_per_w)])

    return k(table, idx)


def kernel(encoder_embedding, W):
    x, w = encoder_embedding, W
    B, D = x.shape
    Dp = -(-D // 128) * 128

    wt = w.T
    xp = jnp.pad(x, ((0, 0), (0, Dp - D)))
    wp = jnp.pad(w, ((0, 0), (0, Dp - D)))

    idx, gap = _stage1(x, wt)
    amb = jnp.nonzero(gap < _TAU, size=_R, fill_value=0)[0].astype(jnp.int32)
    xa = _sc_gather_rows(xp, amb)
    xta = xa[:, :D].T
    idx_full = _stage2(xta, wt, amb, idx)
    out = _sc_gather_rows(wp, idx_full)
    return out[:, :D]
